# baseline (speedup 1.0000x reference)
import jax
import jax.numpy as jnp
from jax.experimental import pallas as pl
from jax.experimental.pallas import tpu as pltpu

_W1 = 4096


def _argmax_zero_kernel(x_ref, zero_ref, idx_ref, rmax_ref, ridx_ref):
    j = pl.program_id(0)
    xb = x_ref[...]
    zero_ref[...] = jnp.zeros_like(zero_ref)
    bmax = jnp.max(xb, axis=1, keepdims=True)
    iota = jax.lax.broadcasted_iota(jnp.int32, xb.shape, 1)
    bidx = jnp.min(
        jnp.where(xb == bmax, iota, xb.shape[1]), axis=1, keepdims=True
    ) + j * _W1

    @pl.when(j == 0)
    def _():
        rmax_ref[...] = bmax
        ridx_ref[...] = bidx

    @pl.when(j > 0)
    def _():
        upd = bmax > rmax_ref[...]
        ridx_ref[...] = jnp.where(upd, bidx, ridx_ref[...])
        rmax_ref[...] = jnp.maximum(bmax, rmax_ref[...])

    @pl.when(j == pl.num_programs(0) - 1)
    def _():
        idx_ref[...] = ridx_ref[...]


def _scatter_kernel(idx_ref, src_ref, dst_ref, eye_ref, sem):
    del src_ref
    rows = dst_ref.shape[0]
    lane = jax.lax.broadcasted_iota(jnp.int32, eye_ref.shape, 1)
    sub = jax.lax.broadcasted_iota(jnp.int32, eye_ref.shape, 0)
    eye_ref[...] = (lane == sub).astype(jnp.float32)

    def _copy(i):
        base = pl.multiple_of((idx_ref[i] // 128) * 128, 128)
        return pltpu.make_async_copy(
            eye_ref.at[pl.ds(idx_ref[i] % 128, 1), :],
            dst_ref.at[pl.ds(i, 1), pl.ds(base, 128)],
            sem,
        )

    def _start(i, carry):
        _copy(i).start()
        return carry

    jax.lax.fori_loop(0, rows, _start, 0)

    def _wait(i, carry):
        _copy(i).wait()
        return carry

    jax.lax.fori_loop(0, rows, _wait, 0)


def kernel(x):
    rows, cols = x.shape
    zeros, idx = pl.pallas_call(
        _argmax_zero_kernel,
        grid=(cols // _W1,),
        in_specs=[pl.BlockSpec((rows, _W1), lambda j: (0, j))],
        out_specs=[
            pl.BlockSpec((rows, _W1), lambda j: (0, j)),
            pl.BlockSpec((rows, 1), lambda j: (0, 0)),
        ],
        out_shape=[
            jax.ShapeDtypeStruct((rows, cols), jnp.float32),
            jax.ShapeDtypeStruct((rows, 1), jnp.int32),
        ],
        scratch_shapes=[
            pltpu.VMEM((rows, 1), jnp.float32),
            pltpu.VMEM((rows, 1), jnp.int32),
        ],
    )(x)
    out = pl.pallas_call(
        _scatter_kernel,
        grid_spec=pltpu.PrefetchScalarGridSpec(
            num_scalar_prefetch=1,
            grid=(1,),
            in_specs=[pl.BlockSpec(memory_space=pltpu.MemorySpace.HBM)],
            out_specs=pl.BlockSpec(memory_space=pltpu.MemorySpace.HBM),
            scratch_shapes=[
                pltpu.VMEM((128, 128), jnp.float32),
                pltpu.SemaphoreType.DMA,
            ],
        ),
        out_shape=jax.ShapeDtypeStruct((rows, cols), jnp.float32),
        input_output_aliases={1: 0},
    )(idx.reshape(rows), zeros)
    return out


# speedup vs baseline: 1.1883x; 1.1883x over previous
import jax
import jax.numpy as jnp
from jax.experimental import pallas as pl
from jax.experimental.pallas import tpu as pltpu

_W = 4096


def _ste_kernel(x_ref, out_ref, rmax_ref, ridx_ref):
    p = pl.program_id(0)
    j = pl.program_id(1)

    @pl.when(p == 0)
    def _():
        xb = x_ref[...]
        bmax = jnp.max(xb, axis=1, keepdims=True)
        iota = jax.lax.broadcasted_iota(jnp.int32, xb.shape, 1)
        bidx = jnp.min(
            jnp.where(xb == bmax, iota, xb.shape[1]), axis=1, keepdims=True
        ) + j * _W

        @pl.when(j == 0)
        def _():
            rmax_ref[...] = bmax
            ridx_ref[...] = bidx

        @pl.when(j > 0)
        def _():
            upd = bmax > rmax_ref[...]
            ridx_ref[...] = jnp.where(upd, bidx, ridx_ref[...])
            rmax_ref[...] = jnp.maximum(bmax, rmax_ref[...])

    @pl.when(p == 1)
    def _():
        iota = jax.lax.broadcasted_iota(jnp.int32, out_ref.shape, 1) + j * _W
        out_ref[...] = (iota == ridx_ref[...]).astype(jnp.float32)


def kernel(x):
    rows, cols = x.shape
    nb = cols // _W
    out = pl.pallas_call(
        _ste_kernel,
        grid=(2, nb),
        in_specs=[
            pl.BlockSpec(
                (rows, _W),
                lambda p, j: (0, jnp.where(p == 0, j, nb - 1)),
            )
        ],
        out_specs=pl.BlockSpec(
            (rows, _W),
            lambda p, j: (0, jnp.where(p == 0, 0, j)),
        ),
        out_shape=jax.ShapeDtypeStruct((rows, cols), jnp.float32),
        scratch_shapes=[
            pltpu.VMEM((rows, 1), jnp.float32),
            pltpu.VMEM((rows, 1), jnp.int32),
        ],
    )(x)
    return out


# speedup vs baseline: 1.4266x; 1.2006x over previous
import jax
import jax.numpy as jnp
from jax.experimental import pallas as pl
from jax.experimental.pallas import tpu as pltpu

_W = 8192


def _ste_kernel(x_ref, out_ref, rmax_ref, ridx_ref):
    p = pl.program_id(0)
    j = pl.program_id(1)

    @pl.when(p == 0)
    def _():
        xb = x_ref[...]
        bmax = jnp.max(xb, axis=1, keepdims=True)
        iota = jax.lax.broadcasted_iota(jnp.int32, xb.shape, 1)
        bidx = jnp.min(
            jnp.where(xb == bmax, iota, xb.shape[1]), axis=1, keepdims=True
        ) + j * _W

        @pl.when(j == 0)
        def _():
            rmax_ref[...] = bmax
            ridx_ref[...] = bidx

        @pl.when(j > 0)
        def _():
            upd = bmax > rmax_ref[...]
            ridx_ref[...] = jnp.where(upd, bidx, ridx_ref[...])
            rmax_ref[...] = jnp.maximum(bmax, rmax_ref[...])

    @pl.when(p == 1)
    def _():
        iota = jax.lax.broadcasted_iota(jnp.int32, out_ref.shape, 1) + j * _W
        out_ref[...] = (iota == ridx_ref[...]).astype(jnp.float32)


def kernel(x):
    rows, cols = x.shape
    nb = cols // _W
    out = pl.pallas_call(
        _ste_kernel,
        grid=(2, nb),
        in_specs=[
            pl.BlockSpec(
                (rows, _W),
                lambda p, j: (0, jnp.where(p == 0, j, nb - 1)),
            )
        ],
        out_specs=pl.BlockSpec(
            (rows, _W),
            lambda p, j: (0, jnp.where(p == 0, 0, j)),
        ),
        out_shape=jax.ShapeDtypeStruct((rows, cols), jnp.float32),
        scratch_shapes=[
            pltpu.VMEM((rows, 1), jnp.float32),
            pltpu.VMEM((rows, 1), jnp.int32),
        ],
    )(x)
    return out
